# baseline (speedup 1.0000x reference)
import functools

import jax
import jax.numpy as jnp
from jax import lax
from jax.experimental import pallas as pl
from jax.experimental.pallas import tpu as pltpu
from jax.experimental.pallas import tpu_sc as plsc

N = 10000
E = 320000
H = 128
L = 4
P = L + 1

ROW_BLK = 1000

NC = 2
NS = 16
NW = NC * NS
EW = E // NW
K = 80
NB = EW // K
NP = 10240
NROWS_T = NP // NS

SQ = 1.0 + 1e-5


def _dot(a, b):
    return jnp.dot(a.astype(jnp.bfloat16), b.astype(jnp.bfloat16),
                   preferred_element_type=jnp.float32)


def _emb_body(h_ref, w_ref, b_ref, o_ref):
    o_ref[...] = _dot(h_ref[...], w_ref[...]) + b_ref[...]


def _tc_embed(h, w, b):
    return pl.pallas_call(
        _emb_body,
        grid=(N // ROW_BLK,),
        in_specs=[
            pl.BlockSpec((ROW_BLK, H), lambda i: (i, 0)),
            pl.BlockSpec((H, H), lambda i: (0, 0)),
            pl.BlockSpec((1, H), lambda i: (0, 0)),
        ],
        out_specs=pl.BlockSpec((ROW_BLK, H), lambda i: (i, 0)),
        out_shape=jax.ShapeDtypeStruct((N, H), jnp.float32),
    )(h, w, b.reshape(1, H))


def _layer_body(x_ref, n_ref, w0_ref, b0_ref, g0_ref, c0_ref,
                w1_ref, b1_ref, ee_ref, ga_ref, ca_ref, gn_ref, cn_ref, o_ref):
    x = x_ref[...]
    t = ee_ref[...] * x + n_ref[...]
    z0 = _dot(t, w0_ref[...]) + b0_ref[...]
    z1 = jnp.maximum(z0 / jnp.sqrt(jnp.float32(SQ)) * g0_ref[...] + c0_ref[...],
                     0.0)
    z2 = _dot(z1, w1_ref[...]) + b1_ref[...]
    y = jnp.maximum(z2 / jnp.sqrt(jnp.float32(SQ)) * ga_ref[...] + ca_ref[...],
                    0.0)
    t2 = y / jnp.sqrt(jnp.float32(SQ)) * gn_ref[...] + cn_ref[...]
    o_ref[...] = x + jnp.maximum(t2, 0.0)


def _tc_layer(x, n0, lp):
    vec = lambda v: v.reshape(1, H)
    return pl.pallas_call(
        _layer_body,
        grid=(N // ROW_BLK,),
        in_specs=[
            pl.BlockSpec((ROW_BLK, H), lambda i: (i, 0)),
            pl.BlockSpec((ROW_BLK, H), lambda i: (i, 0)),
            pl.BlockSpec((H, H), lambda i: (0, 0)),
            pl.BlockSpec((1, H), lambda i: (0, 0)),
            pl.BlockSpec((1, H), lambda i: (0, 0)),
            pl.BlockSpec((1, H), lambda i: (0, 0)),
            pl.BlockSpec((H, H), lambda i: (0, 0)),
            pl.BlockSpec((1, H), lambda i: (0, 0)),
            pl.BlockSpec((1, 1), lambda i: (0, 0)),
            pl.BlockSpec((1, H), lambda i: (0, 0)),
            pl.BlockSpec((1, H), lambda i: (0, 0)),
            pl.BlockSpec((1, H), lambda i: (0, 0)),
            pl.BlockSpec((1, H), lambda i: (0, 0)),
        ],
        out_specs=pl.BlockSpec((ROW_BLK, H), lambda i: (i, 0)),
        out_shape=jax.ShapeDtypeStruct((N, H), jnp.float32),
    )(x, n0, lp['W0'], vec(lp['b0']), vec(lp['bn0_g']), vec(lp['bn0_b']),
      lp['W1'], vec(lp['b1']), (1.0 + lp['eps']).reshape(1, 1),
      vec(lp['bn_apply_g']), vec(lp['bn_apply_b']),
      vec(lp['bn_node_g']), vec(lp['bn_node_b']))


def _pack_body(x_ref, w_ref, b_ref, oa_ref, ob_ref):
    ab = _dot(x_ref[...], w_ref[...]) + b_ref[...]
    oa_ref[...] = ab[:, :H]
    ob_ref[...] = ab[:, H:]


def _tc_pack(x, wab, bab):
    return pl.pallas_call(
        _pack_body,
        grid=(N // ROW_BLK,),
        in_specs=[
            pl.BlockSpec((ROW_BLK, H), lambda i: (i, 0)),
            pl.BlockSpec((H, 2 * H), lambda i: (0, 0)),
            pl.BlockSpec((1, 2 * H), lambda i: (0, 0)),
        ],
        out_specs=[
            pl.BlockSpec((ROW_BLK, H), lambda i: (i, 0)),
            pl.BlockSpec((ROW_BLK, H), lambda i: (i, 0)),
        ],
        out_shape=[
            jax.ShapeDtypeStruct((N, H), jnp.float32),
            jax.ShapeDtypeStruct((N, H), jnp.float32),
        ],
    )(x, wab, bab.reshape(1, 2 * H))


CH = 1600
NCH = E // CH
BR = 128
NBATCH = (CH + 2 * BR + BR - 1) // BR
NROWS_W = NP // NW


def _segsum_body(x_ref, src_ref, dst_ref, out_ref,
                 srcv, dstv, csrc, cdst, rows_v, acc, sem):
    c = lax.axis_index("c")
    s = lax.axis_index("s")
    lo = (c * NS + s) * NROWS_W
    ebase = 0

    iota = lax.iota(jnp.int32, 16)
    zsp = jnp.zeros((16,), jnp.float32)
    zsi = jnp.full((16,), 0, jnp.int32)

    def zacc(r, carry):
        acc[pl.ds(r * 16, 16)] = zsp
        return carry

    lax.fori_loop(0, NROWS_W * H // 16, zacc, 0, unroll=8)

    for r in range(NBATCH):
        for cc in range(BR // 16):
            csrc[r, pl.ds(cc * 16, 16)] = zsi

    def chunk(q, carry):
        cb = ebase + q * CH
        pltpu.sync_copy(src_ref.at[pl.ds(cb, CH)], srcv)
        pltpu.sync_copy(dst_ref.at[pl.ds(cb, CH)], dstv)

        def scan_g(g, cnt):
            dv = dstv[pl.ds(g * 16, 16)]
            sv = srcv[pl.ds(g * 16, 16)]
            dl = dv - lo
            m = (dl >= 0) & (dl < NROWS_W)
            offs = plsc.cumsum(m.astype(jnp.int32))
            idx = cnt + offs - 1
            plsc.store_scatter(csrc, [lax.shift_right_logical(idx, 7), idx & 127],
                               sv, mask=m)
            plsc.store_scatter(cdst, [idx], dl, mask=m)
            return cnt + jnp.sum(m.astype(jnp.int32))

        cnt = lax.fori_loop(0, CH // 16, scan_g, 0, unroll=4)

        dummy = jnp.full((16,), NROWS_W, jnp.int32)
        for p in range(BR // 16):
            plsc.store_scatter(cdst, [cnt + p * 16 + iota], dummy)

        def batch(b, carry2):
            pltpu.async_copy(x_ref.at[csrc.at[b]], rows_v, sem).wait()

            def row(r2, carry3):
                dlv = plsc.load_gather(cdst, [jnp.full((16,), b * BR + r2,
                                                       jnp.int32)])
                ibase = dlv * H + iota
                for cc in range(H // 16):
                    v = rows_v[r2, pl.ds(cc * 16, 16)]
                    cur = plsc.load_gather(acc, [ibase + (cc * 16)])
                    plsc.store_scatter(acc, [ibase + (cc * 16)], cur + v)
                return carry3

            lax.fori_loop(0, BR, row, 0, unroll=2)
            return carry2

        lax.fori_loop(0, (cnt + BR - 1) // BR, batch, 0)
        return carry

    lax.fori_loop(0, NCH, chunk, 0)

    pltpu.sync_copy(acc.at[pl.ds(0, NROWS_W * H)], out_ref.at[c, s])


def _sc_segsum(x, src, dst):
    mesh = plsc.VectorSubcoreMesh(core_axis_name="c", subcore_axis_name="s",
                                  num_cores=NC, num_subcores=NS)
    f = pl.kernel(
        _segsum_body,
        out_type=jax.ShapeDtypeStruct((NC, NS, NROWS_W * H), jnp.float32),
        mesh=mesh,
        scratch_types=[
            pltpu.VMEM((CH,), jnp.int32),
            pltpu.VMEM((CH,), jnp.int32),
            pltpu.VMEM((NBATCH, BR), jnp.int32),
            pltpu.VMEM((CH + 2 * BR,), jnp.int32),
            pltpu.VMEM((BR, H), jnp.float32),
            pltpu.VMEM(((NROWS_W + 1) * H,), jnp.float32),
            pltpu.SemaphoreType.DMA,
        ],
        compiler_params=pltpu.CompilerParams(needs_layout_passes=False),
    )
    out = f(x, src, dst)
    return out.reshape(NP, H)


W_U = 0
W_V = P * H
W_W2 = 2 * P * H
W_ACCI = 3 * P * H
W_LEN = 3 * P * H + 16

SJ = 2000
NSJ = EW // SJ
KB = 40
NBJ = SJ // KB


def _rnd16(v):
    i = plsc.bitcast(v, jnp.int32)
    lsb = lax.shift_right_logical(i, 16) & 1
    r = (i + 0x7FFF + lsb) & jnp.int32(-65536)
    return plsc.bitcast(r, jnp.float32)


def _scorer_body(ap_ref, bp_ref, src_ref, dst_ref, e_ref, w_ref, out_ref,
                 idxs_sc, idxd_sc, e_sc, rA0, rB0, rA1, rB1, acc_v, wts_v,
                 out_v, semr0, semr1):
    c = lax.axis_index("c")
    s = lax.axis_index("s")
    wid = s * NC + c
    base = wid * EW

    pltpu.sync_copy(w_ref, wts_v)
    acci = wts_v[pl.ds(W_ACCI, 16)]
    col0 = jnp.full((16,), 0, jnp.int32)
    col1 = jnp.full((16,), 1, jnp.int32)
    lane0 = lax.iota(jnp.int32, 16) == col0

    uvw = []
    for l in range(P):
        us = [wts_v[pl.ds(W_U + l * H + cc * 16, 16)] for cc in range(H // 16)]
        vs = [wts_v[pl.ds(W_V + l * H + cc * 16, 16)] for cc in range(H // 16)]
        ws = [wts_v[pl.ds(W_W2 + l * H + cc * 16, 16)] for cc in range(H // 16)]
        uvw.append((us, vs, ws))

    def issue(jj, rA, rB, sem):
        pltpu.async_copy(ap_ref.at[idxs_sc.at[pl.ds(jj * KB, KB)]], rA, sem)
        pltpu.async_copy(bp_ref.at[idxd_sc.at[pl.ds(jj * KB, KB)]], rB, sem)

    def wait_pair(rA, rB, sem):
        pltpu.make_async_copy(ap_ref.at[idxs_sc.at[pl.ds(0, KB)]], rA, sem).wait()
        pltpu.make_async_copy(bp_ref.at[idxd_sc.at[pl.ds(0, KB)]], rB, sem).wait()

    def compute(jj, sbase, rA, rB):
        @plsc.parallel_loop(0, KB, unroll=2)
        def init_i(i):
            acc_v[i, :] = acci

        for l in range(P):
            us, vs, ws = uvw[l]

            @plsc.parallel_loop(0, KB, unroll=2)
            def edge_i(i, l=l, us=us, vs=vs, ws=ws):
                si = jnp.full((16,), 2 * (jj * KB + i), jnp.int32)
                e0 = plsc.load_gather(e_sc, [si + col0])
                e1 = plsc.load_gather(e_sc, [si + col1])
                acc = acc_v[i, :]
                for cc in range(H // 16):
                    a = rA[i, pl.ds(l * H + cc * 16, 16)]
                    b = rB[i, pl.ds(l * H + cc * 16, 16)]
                    sv = a + b + e0 * us[cc] + e1 * vs[cc]
                    svr = _rnd16(jnp.maximum(sv, 0.0))
                    acc = acc + svr * ws[cc]
                acc_v[i, :] = acc

        @plsc.parallel_loop(0, KB, unroll=2)
        def fin_i(i):
            ssum = jnp.sum(acc_v[i, :])
            val = jnp.full((16,), jnp.maximum(ssum, 0.0), jnp.float32)
            si = jnp.full((16,), i, jnp.int32)
            plsc.store_scatter(out_v, [si], val, mask=lane0)

        pltpu.sync_copy(out_v, out_ref.at[pl.ds(sbase + jj * KB, KB)])

    def superchunk(sj, carry):
        sbase = base + sj * SJ
        pltpu.sync_copy(src_ref.at[pl.ds(sbase, SJ)], idxs_sc)
        pltpu.sync_copy(dst_ref.at[pl.ds(sbase, SJ)], idxd_sc)
        pltpu.sync_copy(e_ref.at[pl.ds(2 * sbase, 2 * SJ)], e_sc)
        issue(0, rA0, rB0, semr0)

        def two_blocks(t, carry2):
            j0 = 2 * t
            j1 = 2 * t + 1
            issue(j1, rA1, rB1, semr1)
            wait_pair(rA0, rB0, semr0)
            compute(j0, sbase, rA0, rB0)
            j2 = (2 * t + 2) % NBJ
            issue(j2, rA0, rB0, semr0)
            wait_pair(rA1, rB1, semr1)
            compute(j1, sbase, rA1, rB1)
            return carry2

        lax.fori_loop(0, NBJ // 2, two_blocks, 0)
        wait_pair(rA0, rB0, semr0)
        return carry

    lax.fori_loop(0, NSJ, superchunk, 0)


def _sc_score(apack, bpack, src, dst, e, wts):
    mesh = plsc.VectorSubcoreMesh(core_axis_name="c", subcore_axis_name="s",
                                  num_cores=NC, num_subcores=NS)
    f = pl.kernel(
        _scorer_body,
        out_type=jax.ShapeDtypeStruct((E,), jnp.float32),
        mesh=mesh,
        scratch_types=[
            pltpu.VMEM((SJ,), jnp.int32),
            pltpu.VMEM((SJ,), jnp.int32),
            pltpu.VMEM((2 * SJ,), jnp.float32),
            pltpu.VMEM((KB, P * H), jnp.float32),
            pltpu.VMEM((KB, P * H), jnp.float32),
            pltpu.VMEM((KB, P * H), jnp.float32),
            pltpu.VMEM((KB, P * H), jnp.float32),
            pltpu.VMEM((KB, 16), jnp.float32),
            pltpu.VMEM((W_LEN,), jnp.float32),
            pltpu.VMEM((KB,), jnp.float32),
            pltpu.SemaphoreType.DMA,
            pltpu.SemaphoreType.DMA,
        ],
        compiler_params=pltpu.CompilerParams(needs_layout_passes=False),
    )
    return f(apack, bpack, src, dst, e.reshape(E * 2), wts)


def kernel(h, edge_index, e, params):
    src = edge_index[0]
    dst = edge_index[1]

    x = _tc_embed(h, params['emb_W'], params['emb_b'])

    xs = [x]
    for lp in params['layers']:
        neigh = _sc_segsum(x, src, dst)
        x = _tc_layer(x, neigh[:N], lp)
        xs.append(x)

    aparts, bparts = [], []
    for xl, pp in zip(xs, params['preds']):
        wab = jnp.concatenate([pp['W1'][:H], pp['W1'][H:2 * H]], axis=1)
        bab = jnp.concatenate([pp['b1'], jnp.zeros((H,), jnp.float32)])
        al, bl = _tc_pack(xl, wab, bab)
        aparts.append(al)
        bparts.append(bl)
    apack = jnp.concatenate(aparts, axis=1)
    bpack = jnp.concatenate(bparts, axis=1)

    rnd = lambda w: w.astype(jnp.bfloat16).astype(jnp.float32)
    us = jnp.concatenate([rnd(pp['W1'][2 * H]) for pp in params['preds']])
    vs = jnp.concatenate([rnd(pp['W1'][2 * H + 1]) for pp in params['preds']])
    w2 = jnp.concatenate([rnd(pp['W2'][:, 0]) for pp in params['preds']])
    acci = jnp.zeros((16,), jnp.float32).at[0].set(
        sum(pp['b2'][0] for pp in params['preds']))
    wts = jnp.concatenate([us, vs, w2, acci])

    score = _sc_score(apack, bpack, src, dst, rnd(e), wts)
    return score[:, None]


# speedup vs baseline: 5.3892x; 5.3892x over previous
import functools

import jax
import jax.numpy as jnp
from jax import lax
from jax.experimental import pallas as pl
from jax.experimental.pallas import tpu as pltpu
from jax.experimental.pallas import tpu_sc as plsc

N = 10000
E = 320000
H = 128
L = 4
P = L + 1

ROW_BLK = 1000

NC = 2
NS = 16
NW = NC * NS
EW = E // NW
K = 80
NB = EW // K
NP = 10240
NROWS_T = NP // NS

SQ = 1.0 + 1e-5


def _dot(a, b):
    return jnp.dot(a.astype(jnp.bfloat16), b.astype(jnp.bfloat16),
                   preferred_element_type=jnp.float32)


def _emb_body(h_ref, w_ref, b_ref, o_ref):
    o_ref[...] = _dot(h_ref[...], w_ref[...]) + b_ref[...]


def _tc_embed(h, w, b):
    return pl.pallas_call(
        _emb_body,
        grid=(N // ROW_BLK,),
        in_specs=[
            pl.BlockSpec((ROW_BLK, H), lambda i: (i, 0)),
            pl.BlockSpec((H, H), lambda i: (0, 0)),
            pl.BlockSpec((1, H), lambda i: (0, 0)),
        ],
        out_specs=pl.BlockSpec((ROW_BLK, H), lambda i: (i, 0)),
        out_shape=jax.ShapeDtypeStruct((N, H), jnp.float32),
    )(h, w, b.reshape(1, H))


def _layer_body(x_ref, n_ref, w0_ref, b0_ref, g0_ref, c0_ref,
                w1_ref, b1_ref, ee_ref, ga_ref, ca_ref, gn_ref, cn_ref, o_ref):
    x = x_ref[...]
    t = ee_ref[...] * x + n_ref[...]
    z0 = _dot(t, w0_ref[...]) + b0_ref[...]
    z1 = jnp.maximum(z0 / jnp.sqrt(jnp.float32(SQ)) * g0_ref[...] + c0_ref[...],
                     0.0)
    z2 = _dot(z1, w1_ref[...]) + b1_ref[...]
    y = jnp.maximum(z2 / jnp.sqrt(jnp.float32(SQ)) * ga_ref[...] + ca_ref[...],
                    0.0)
    t2 = y / jnp.sqrt(jnp.float32(SQ)) * gn_ref[...] + cn_ref[...]
    o_ref[...] = x + jnp.maximum(t2, 0.0)


def _tc_layer(x, n0, lp):
    vec = lambda v: v.reshape(1, H)
    return pl.pallas_call(
        _layer_body,
        grid=(N // ROW_BLK,),
        in_specs=[
            pl.BlockSpec((ROW_BLK, H), lambda i: (i, 0)),
            pl.BlockSpec((ROW_BLK, H), lambda i: (i, 0)),
            pl.BlockSpec((H, H), lambda i: (0, 0)),
            pl.BlockSpec((1, H), lambda i: (0, 0)),
            pl.BlockSpec((1, H), lambda i: (0, 0)),
            pl.BlockSpec((1, H), lambda i: (0, 0)),
            pl.BlockSpec((H, H), lambda i: (0, 0)),
            pl.BlockSpec((1, H), lambda i: (0, 0)),
            pl.BlockSpec((1, 1), lambda i: (0, 0)),
            pl.BlockSpec((1, H), lambda i: (0, 0)),
            pl.BlockSpec((1, H), lambda i: (0, 0)),
            pl.BlockSpec((1, H), lambda i: (0, 0)),
            pl.BlockSpec((1, H), lambda i: (0, 0)),
        ],
        out_specs=pl.BlockSpec((ROW_BLK, H), lambda i: (i, 0)),
        out_shape=jax.ShapeDtypeStruct((N, H), jnp.float32),
    )(x, n0, lp['W0'], vec(lp['b0']), vec(lp['bn0_g']), vec(lp['bn0_b']),
      lp['W1'], vec(lp['b1']), (1.0 + lp['eps']).reshape(1, 1),
      vec(lp['bn_apply_g']), vec(lp['bn_apply_b']),
      vec(lp['bn_node_g']), vec(lp['bn_node_b']))


def _pack_body(x_ref, w_ref, b_ref, oa_ref, ob_ref):
    ab = _dot(x_ref[...], w_ref[...]) + b_ref[...]
    oa_ref[...] = ab[:, :H]
    ob_ref[...] = ab[:, H:]


def _tc_pack(x, wab, bab):
    return pl.pallas_call(
        _pack_body,
        grid=(N // ROW_BLK,),
        in_specs=[
            pl.BlockSpec((ROW_BLK, H), lambda i: (i, 0)),
            pl.BlockSpec((H, 2 * H), lambda i: (0, 0)),
            pl.BlockSpec((1, 2 * H), lambda i: (0, 0)),
        ],
        out_specs=[
            pl.BlockSpec((ROW_BLK, H), lambda i: (i, 0)),
            pl.BlockSpec((ROW_BLK, H), lambda i: (i, 0)),
        ],
        out_shape=[
            jax.ShapeDtypeStruct((N, H), jnp.float32),
            jax.ShapeDtypeStruct((N, H), jnp.float32),
        ],
    )(x, wab, bab.reshape(1, 2 * H))


CH = 8000
NCH = E // CH
BR = 128
NBATCH = (CH + 2 * BR + BR - 1) // BR
NROWS_W = NP // NW


def _segsum_body(x_ref, src_ref, dst_ref, out_ref,
                 srcv, dstv, csrc, cdst, rows_v, acc, sem):
    c = lax.axis_index("c")
    s = lax.axis_index("s")
    lo = (c * NS + s) * NROWS_W
    ebase = 0

    iota = lax.iota(jnp.int32, 16)
    zsp = jnp.zeros((16,), jnp.float32)
    zsi = jnp.full((16,), 0, jnp.int32)

    def zacc(r, carry):
        acc[pl.ds(r * 16, 16)] = zsp
        return carry

    lax.fori_loop(0, NROWS_W * H // 16, zacc, 0, unroll=8)

    for r in range(NBATCH):
        for cc in range(BR // 16):
            csrc[r, pl.ds(cc * 16, 16)] = zsi

    def chunk(q, carry):
        cb = ebase + q * CH
        pltpu.sync_copy(src_ref.at[pl.ds(cb, CH)], srcv)
        pltpu.sync_copy(dst_ref.at[pl.ds(cb, CH)], dstv)

        def scan_g(g, cnt):
            dv = dstv[pl.ds(g * 16, 16)]
            sv = srcv[pl.ds(g * 16, 16)]
            dl = dv - lo
            m = (dl >= 0) & (dl < NROWS_W)
            offs = plsc.cumsum(m.astype(jnp.int32))
            idx = cnt + offs - 1
            plsc.store_scatter(csrc, [lax.shift_right_logical(idx, 7), idx & 127],
                               sv, mask=m)
            plsc.store_scatter(cdst, [idx], dl, mask=m)
            return cnt + jnp.sum(m.astype(jnp.int32))

        cnt = lax.fori_loop(0, CH // 16, scan_g, 0, unroll=4)

        dummy = jnp.full((16,), NROWS_W, jnp.int32)
        for p in range(BR // 16):
            plsc.store_scatter(cdst, [cnt + p * 16 + iota], dummy)

        def batch(b, carry2):
            pltpu.async_copy(x_ref.at[csrc.at[b]], rows_v, sem).wait()

            def row(r2, carry3):
                dlv = plsc.load_gather(cdst, [jnp.full((16,), b * BR + r2,
                                                       jnp.int32)])
                ibase = dlv * H + iota
                for cc in range(H // 16):
                    v = rows_v[r2, pl.ds(cc * 16, 16)]
                    cur = plsc.load_gather(acc, [ibase + (cc * 16)])
                    plsc.store_scatter(acc, [ibase + (cc * 16)], cur + v)
                return carry3

            lax.fori_loop(0, BR, row, 0, unroll=2)
            return carry2

        lax.fori_loop(0, (cnt + BR - 1) // BR, batch, 0)
        return carry

    lax.fori_loop(0, NCH, chunk, 0)

    pltpu.sync_copy(acc.at[pl.ds(0, NROWS_W * H)], out_ref.at[c, s])


def _sc_segsum(x, src, dst):
    mesh = plsc.VectorSubcoreMesh(core_axis_name="c", subcore_axis_name="s",
                                  num_cores=NC, num_subcores=NS)
    f = pl.kernel(
        _segsum_body,
        out_type=jax.ShapeDtypeStruct((NC, NS, NROWS_W * H), jnp.float32),
        mesh=mesh,
        scratch_types=[
            pltpu.VMEM((CH,), jnp.int32),
            pltpu.VMEM((CH,), jnp.int32),
            pltpu.VMEM((NBATCH, BR), jnp.int32),
            pltpu.VMEM((CH + 2 * BR,), jnp.int32),
            pltpu.VMEM((BR, H), jnp.float32),
            pltpu.VMEM(((NROWS_W + 1) * H,), jnp.float32),
            pltpu.SemaphoreType.DMA,
        ],
        compiler_params=pltpu.CompilerParams(needs_layout_passes=False),
    )
    out = f(x, src, dst)
    return out.reshape(NP, H)


W_U = 0
W_V = P * H
W_W2 = 2 * P * H
W_ACCI = 3 * P * H
W_LEN = 3 * P * H + 16

SJ = 2000
NSJ = EW // SJ
KB = 40
NBJ = SJ // KB


def _rnd16(v):
    i = plsc.bitcast(v, jnp.int32)
    lsb = lax.shift_right_logical(i, 16) & 1
    r = (i + 0x7FFF + lsb) & jnp.int32(-65536)
    return plsc.bitcast(r, jnp.float32)


def _scorer_body(ap_ref, bp_ref, src_ref, dst_ref, e_ref, w_ref, out_ref,
                 idxs_sc, idxd_sc, e_sc, rA0, rB0, rA1, rB1, acc_v, wts_v,
                 out_v, semr0, semr1):
    c = lax.axis_index("c")
    s = lax.axis_index("s")
    wid = s * NC + c
    base = wid * EW

    pltpu.sync_copy(w_ref, wts_v)
    acci = wts_v[pl.ds(W_ACCI, 16)]
    col0 = jnp.full((16,), 0, jnp.int32)
    col1 = jnp.full((16,), 1, jnp.int32)
    lane0 = lax.iota(jnp.int32, 16) == col0

    uvw = []
    for l in range(P):
        us = [wts_v[pl.ds(W_U + l * H + cc * 16, 16)] for cc in range(H // 16)]
        vs = [wts_v[pl.ds(W_V + l * H + cc * 16, 16)] for cc in range(H // 16)]
        ws = [wts_v[pl.ds(W_W2 + l * H + cc * 16, 16)] for cc in range(H // 16)]
        uvw.append((us, vs, ws))

    def issue(jj, rA, rB, sem):
        pltpu.async_copy(ap_ref.at[idxs_sc.at[pl.ds(jj * KB, KB)]], rA, sem)
        pltpu.async_copy(bp_ref.at[idxd_sc.at[pl.ds(jj * KB, KB)]], rB, sem)

    def wait_pair(rA, rB, sem):
        pltpu.make_async_copy(ap_ref.at[idxs_sc.at[pl.ds(0, KB)]], rA, sem).wait()
        pltpu.make_async_copy(bp_ref.at[idxd_sc.at[pl.ds(0, KB)]], rB, sem).wait()

    def compute(jj, sbase, rA, rB):
        @plsc.parallel_loop(0, KB, unroll=2)
        def init_i(i):
            acc_v[i, :] = acci

        for l in range(P):
            us, vs, ws = uvw[l]

            @plsc.parallel_loop(0, KB, unroll=2)
            def edge_i(i, l=l, us=us, vs=vs, ws=ws):
                si = jnp.full((16,), 2 * (jj * KB + i), jnp.int32)
                e0 = plsc.load_gather(e_sc, [si + col0])
                e1 = plsc.load_gather(e_sc, [si + col1])
                acc = acc_v[i, :]
                for cc in range(H // 16):
                    a = rA[i, pl.ds(l * H + cc * 16, 16)]
                    b = rB[i, pl.ds(l * H + cc * 16, 16)]
                    sv = a + b + e0 * us[cc] + e1 * vs[cc]
                    svr = _rnd16(jnp.maximum(sv, 0.0))
                    acc = acc + svr * ws[cc]
                acc_v[i, :] = acc

        @plsc.parallel_loop(0, KB, unroll=2)
        def fin_i(i):
            ssum = jnp.sum(acc_v[i, :])
            val = jnp.full((16,), jnp.maximum(ssum, 0.0), jnp.float32)
            si = jnp.full((16,), i, jnp.int32)
            plsc.store_scatter(out_v, [si], val, mask=lane0)

        pltpu.sync_copy(out_v, out_ref.at[pl.ds(sbase + jj * KB, KB)])

    def superchunk(sj, carry):
        sbase = base + sj * SJ
        pltpu.sync_copy(src_ref.at[pl.ds(sbase, SJ)], idxs_sc)
        pltpu.sync_copy(dst_ref.at[pl.ds(sbase, SJ)], idxd_sc)
        pltpu.sync_copy(e_ref.at[pl.ds(2 * sbase, 2 * SJ)], e_sc)
        issue(0, rA0, rB0, semr0)

        def two_blocks(t, carry2):
            j0 = 2 * t
            j1 = 2 * t + 1
            issue(j1, rA1, rB1, semr1)
            wait_pair(rA0, rB0, semr0)
            compute(j0, sbase, rA0, rB0)
            j2 = (2 * t + 2) % NBJ
            issue(j2, rA0, rB0, semr0)
            wait_pair(rA1, rB1, semr1)
            compute(j1, sbase, rA1, rB1)
            return carry2

        lax.fori_loop(0, NBJ // 2, two_blocks, 0)
        wait_pair(rA0, rB0, semr0)
        return carry

    lax.fori_loop(0, NSJ, superchunk, 0)


def _sc_score(apack, bpack, src, dst, e, wts):
    mesh = plsc.VectorSubcoreMesh(core_axis_name="c", subcore_axis_name="s",
                                  num_cores=NC, num_subcores=NS)
    f = pl.kernel(
        _scorer_body,
        out_type=jax.ShapeDtypeStruct((E,), jnp.float32),
        mesh=mesh,
        scratch_types=[
            pltpu.VMEM((SJ,), jnp.int32),
            pltpu.VMEM((SJ,), jnp.int32),
            pltpu.VMEM((2 * SJ,), jnp.float32),
            pltpu.VMEM((KB, P * H), jnp.float32),
            pltpu.VMEM((KB, P * H), jnp.float32),
            pltpu.VMEM((KB, P * H), jnp.float32),
            pltpu.VMEM((KB, P * H), jnp.float32),
            pltpu.VMEM((KB, 16), jnp.float32),
            pltpu.VMEM((W_LEN,), jnp.float32),
            pltpu.VMEM((KB,), jnp.float32),
            pltpu.SemaphoreType.DMA,
            pltpu.SemaphoreType.DMA,
        ],
        compiler_params=pltpu.CompilerParams(needs_layout_passes=False),
    )
    return f(apack, bpack, src, dst, e.reshape(E * 2), wts)


def kernel(h, edge_index, e, params):
    src = edge_index[0]
    dst = edge_index[1]

    x = _tc_embed(h, params['emb_W'], params['emb_b'])

    xs = [x]
    for lp in params['layers']:
        neigh = _sc_segsum(x, src, dst)
        x = _tc_layer(x, neigh[:N], lp)
        xs.append(x)

    aparts, bparts = [], []
    for xl, pp in zip(xs, params['preds']):
        wab = jnp.concatenate([pp['W1'][:H], pp['W1'][H:2 * H]], axis=1)
        bab = jnp.concatenate([pp['b1'], jnp.zeros((H,), jnp.float32)])
        al, bl = _tc_pack(xl, wab, bab)
        aparts.append(al)
        bparts.append(bl)
    apack = jnp.concatenate(aparts, axis=1)
    bpack = jnp.concatenate(bparts, axis=1)

    rnd = lambda w: w.astype(jnp.bfloat16).astype(jnp.float32)
    us = jnp.concatenate([rnd(pp['W1'][2 * H]) for pp in params['preds']])
    vs = jnp.concatenate([rnd(pp['W1'][2 * H + 1]) for pp in params['preds']])
    w2 = jnp.concatenate([rnd(pp['W2'][:, 0]) for pp in params['preds']])
    acci = jnp.zeros((16,), jnp.float32).at[0].set(
        sum(pp['b2'][0] for pp in params['preds']))
    wts = jnp.concatenate([us, vs, w2, acci])

    score = _sc_score(apack, bpack, src, dst, rnd(e), wts)
    return score[:, None]
